# Initial kernel scaffold; baseline (speedup 1.0000x reference)
import functools

import jax
import jax.numpy as jnp
from jax import lax
from jax.experimental import pallas as pl
from jax.experimental.pallas import tpu as pltpu
from jax.experimental.pallas import tpu_sc as plsc

_B = 16384
_F = 26
_K = 32
_NW = 32
_RW = _B // _NW
_NCH = 16
_RC = _RW // _NCH
_GPC = 8
_GI = _RC * _F // _GPC
_LANES = 16


def _issue(c, idx_v, V_hbm, w_hbm, gbuf, wbuf, sem):
    for j in range(_GPC):
        g = c * _GPC + j
        pltpu.async_copy(V_hbm.at[idx_v.at[g]], gbuf.at[pl.ds(j * _GI, _GI)], sem)
        pltpu.async_copy(w_hbm.at[idx_v.at[g]], wbuf.at[pl.ds(j * _GI, _GI)], sem)


def _drain(c, idx_v, V_hbm, w_hbm, gbuf, wbuf, sem):
    for j in range(_GPC):
        g = c * _GPC + j
        pltpu.make_async_copy(
            V_hbm.at[idx_v.at[g]], gbuf.at[pl.ds(j * _GI, _GI)], sem).wait()
        pltpu.make_async_copy(
            w_hbm.at[idx_v.at[g]], wbuf.at[pl.ds(j * _GI, _GI)], sem).wait()


def _compute(c, gbuf, wbuf, outv):
    lanes = lax.iota(jnp.int32, _LANES)

    def row_body(r, carry):
        base = r * _F
        acc0 = jnp.zeros((_LANES,), jnp.float32)
        acc1 = jnp.zeros((_LANES,), jnp.float32)
        q0 = jnp.zeros((_LANES,), jnp.float32)
        q1 = jnp.zeros((_LANES,), jnp.float32)
        for f in range(_F):
            v0 = gbuf[base + f, pl.ds(0, _LANES)]
            v1 = gbuf[base + f, pl.ds(_LANES, _LANES)]
            acc0 = acc0 + v0
            q0 = q0 + v0 * v0
            acc1 = acc1 + v1
            q1 = q1 + v1 * v1
        d = (acc0 * acc0 - q0) + (acc1 * acc1 - q1)
        t = jnp.sum(d) * jnp.float32(0.5)
        wv0 = wbuf[pl.ds(base, _LANES)]
        wv1 = wbuf[pl.ds(base + _LANES, _LANES)]
        ws = jnp.sum(wv0) + jnp.sum(
            jnp.where(lanes < _F - _LANES, wv1, jnp.float32(0.0)))
        tot = t + ws
        grp = (r // _LANES) * _LANES
        slot = pl.ds(c * _RC + grp, _LANES)
        prev = outv[slot]
        outv[slot] = jnp.where(lanes == r % _LANES, tot, prev)
        return carry

    lax.fori_loop(0, _RC, row_body, 0)


def _fm_body(x_hbm, V_hbm, w_hbm, out_hbm,
             idx_v, g0, g1, wb0, wb1, outv, sem0, sem1):
    info = plsc.get_sparse_core_info()
    wid = lax.axis_index("s") * info.num_cores + lax.axis_index("c")
    pltpu.sync_copy(x_hbm.at[wid], idx_v)
    _issue(0, idx_v, V_hbm, w_hbm, g0, wb0, sem0)

    def body(s, carry):
        c0 = 2 * s
        c1 = 2 * s + 1
        _issue(c1, idx_v, V_hbm, w_hbm, g1, wb1, sem1)
        _drain(c0, idx_v, V_hbm, w_hbm, g0, wb0, sem0)
        _compute(c0, g0, wb0, outv)

        @pl.when(s < _NCH // 2 - 1)
        def _():
            _issue(c0 + 2, idx_v, V_hbm, w_hbm, g0, wb0, sem0)

        _drain(c1, idx_v, V_hbm, w_hbm, g1, wb1, sem1)
        _compute(c1, g1, wb1, outv)
        return carry

    lax.fori_loop(0, _NCH // 2, body, 0)
    pltpu.sync_copy(outv, out_hbm.at[pl.ds(wid * _RW, _RW)])


@jax.jit
def _fm(xr, V, w):
    mesh = plsc.VectorSubcoreMesh(core_axis_name="c", subcore_axis_name="s")
    run = functools.partial(
        pl.kernel,
        out_type=jax.ShapeDtypeStruct((_B,), jnp.float32),
        mesh=mesh,
        scratch_types=[
            pltpu.VMEM((_NCH * _GPC, _GI), jnp.int32),
            pltpu.VMEM((_RC * _F, _K), jnp.float32),
            pltpu.VMEM((_RC * _F, _K), jnp.float32),
            pltpu.VMEM((_RC * _F + _LANES,), jnp.float32),
            pltpu.VMEM((_RC * _F + _LANES,), jnp.float32),
            pltpu.VMEM((_RW,), jnp.float32),
            pltpu.SemaphoreType.DMA,
            pltpu.SemaphoreType.DMA,
        ],
    )(_fm_body)
    return run(xr, V, w)


def kernel(x, V, w, w0):
    xr = x.reshape(_NW, _NCH * _GPC, _GI)
    return _fm(xr, V, w) + w0


# speedup vs baseline: 2.1678x; 2.1678x over previous
import functools

import jax
import jax.numpy as jnp
from jax import lax
from jax.experimental import pallas as pl
from jax.experimental.pallas import tpu as pltpu
from jax.experimental.pallas import tpu_sc as plsc

_B = 16384
_F = 26
_K = 32
_NW = 32
_RW = _B // _NW
_NCH = 16
_RC = _RW // _NCH
_GPC = 8
_GI = _RC * _F // _GPC
_LANES = 16


def _issue(c, idx_v, V_hbm, w_hbm, gbuf, wbuf, sem):
    for j in range(_GPC):
        g = c * _GPC + j
        pltpu.async_copy(V_hbm.at[idx_v.at[g]], gbuf.at[pl.ds(j * _GI, _GI)], sem)
        pltpu.async_copy(w_hbm.at[idx_v.at[g]], wbuf.at[pl.ds(j * _GI, _GI)], sem)


def _drain(c, idx_v, V_hbm, w_hbm, gbuf, wbuf, sem):
    for j in range(_GPC):
        g = c * _GPC + j
        pltpu.make_async_copy(
            V_hbm.at[idx_v.at[g]], gbuf.at[pl.ds(j * _GI, _GI)], sem).wait()
        pltpu.make_async_copy(
            w_hbm.at[idx_v.at[g]], wbuf.at[pl.ds(j * _GI, _GI)], sem).wait()


def _compute(c, gbuf, wbuf, outv):
    lanes = lax.iota(jnp.int32, _LANES)

    def row_body(r, carry):
        base = r * _F
        acc0 = jnp.zeros((_LANES,), jnp.float32)
        acc1 = jnp.zeros((_LANES,), jnp.float32)
        q0 = jnp.zeros((_LANES,), jnp.float32)
        q1 = jnp.zeros((_LANES,), jnp.float32)
        for f in range(_F):
            v0 = gbuf[base + f, pl.ds(0, _LANES)]
            v1 = gbuf[base + f, pl.ds(_LANES, _LANES)]
            acc0 = acc0 + v0
            q0 = q0 + v0 * v0
            acc1 = acc1 + v1
            q1 = q1 + v1 * v1
        d = (acc0 * acc0 - q0) + (acc1 * acc1 - q1)
        t = jnp.sum(d) * jnp.float32(0.5)
        wv0 = wbuf[pl.ds(base, _LANES)]
        wv1 = wbuf[pl.ds(base + _LANES, _LANES)]
        ws = jnp.sum(wv0) + jnp.sum(
            jnp.where(lanes < _F - _LANES, wv1, jnp.float32(0.0)))
        tot = t + ws
        grp = (r // _LANES) * _LANES
        slot = pl.ds(c * _RC + grp, _LANES)
        prev = outv[slot]
        outv[slot] = jnp.where(lanes == r % _LANES, tot, prev)
        return carry

    lax.fori_loop(0, _RC, row_body, 0)


def _fm_body(x_hbm, V_hbm, w_hbm, out_hbm,
             idx_v, g0, g1, wb0, wb1, outv, sem0, sem1):
    info = plsc.get_sparse_core_info()
    wid = lax.axis_index("s") * info.num_cores + lax.axis_index("c")
    pltpu.sync_copy(x_hbm.at[wid], idx_v)
    _issue(0, idx_v, V_hbm, w_hbm, g0, wb0, sem0)

    def body(s, carry):
        c0 = 2 * s
        c1 = 2 * s + 1
        _issue(c1, idx_v, V_hbm, w_hbm, g1, wb1, sem1)
        _drain(c0, idx_v, V_hbm, w_hbm, g0, wb0, sem0)
        _compute(c0, g0, wb0, outv)

        @pl.when(s < _NCH // 2 - 1)
        def _():
            _issue(c0 + 2, idx_v, V_hbm, w_hbm, g0, wb0, sem0)

        _drain(c1, idx_v, V_hbm, w_hbm, g1, wb1, sem1)
        _compute(c1, g1, wb1, outv)
        return carry

    lax.fori_loop(0, _NCH // 2, body, 0)
    pltpu.sync_copy(outv, out_hbm.at[pl.ds(wid * _RW, _RW)])


@jax.jit
def _fm(xr, V, w):
    mesh = plsc.VectorSubcoreMesh(core_axis_name="c", subcore_axis_name="s")
    run = functools.partial(
        pl.kernel,
        out_type=jax.ShapeDtypeStruct((_B,), jnp.float32),
        mesh=mesh,
        compiler_params=pltpu.CompilerParams(
            needs_layout_passes=False, use_tc_tiling_on_sc=False),
        scratch_types=[
            pltpu.VMEM((_NCH * _GPC, _GI), jnp.int32),
            pltpu.VMEM((_RC * _F, _K), jnp.float32),
            pltpu.VMEM((_RC * _F, _K), jnp.float32),
            pltpu.VMEM((_RC * _F + _LANES,), jnp.float32),
            pltpu.VMEM((_RC * _F + _LANES,), jnp.float32),
            pltpu.VMEM((_RW,), jnp.float32),
            pltpu.SemaphoreType.DMA,
            pltpu.SemaphoreType.DMA,
        ],
    )(_fm_body)
    return run(xr, V, w)


def kernel(x, V, w, w0):
    xr = x.reshape(_NW, _NCH * _GPC, _GI)
    return _fm(xr, V, w) + w0


# speedup vs baseline: 2.1893x; 1.0099x over previous
import functools

import jax
import jax.numpy as jnp
from jax import lax
from jax.experimental import pallas as pl
from jax.experimental.pallas import tpu as pltpu
from jax.experimental.pallas import tpu_sc as plsc

_B = 16384
_F = 26
_K = 32
_NW = 32
_RW = _B // _NW
_NCH = 16
_RC = _RW // _NCH
_LANES = 16


def _issue(c, idx_v, V_hbm, w_hbm, gbuf, wbuf, sem):
    for f in range(_F):
        idx = idx_v.at[f, pl.ds(c * _RC, _RC)]
        pltpu.async_copy(V_hbm.at[idx], gbuf.at[pl.ds(f * _RC, _RC)], sem)
        pltpu.async_copy(w_hbm.at[idx], wbuf.at[pl.ds(f * _RC, _RC)], sem)


def _drain(c, idx_v, V_hbm, w_hbm, gbuf, wbuf, sem):
    for f in range(_F):
        idx = idx_v.at[f, pl.ds(c * _RC, _RC)]
        pltpu.make_async_copy(
            V_hbm.at[idx], gbuf.at[pl.ds(f * _RC, _RC)], sem).wait()
        pltpu.make_async_copy(
            w_hbm.at[idx], wbuf.at[pl.ds(f * _RC, _RC)], sem).wait()


def _compute(c, gbuf, wbuf, outv):
    lanes = lax.iota(jnp.int32, _LANES)
    zeros = jnp.zeros((_LANES,), jnp.float32)
    for h in range(_RC // _LANES):
        wsum = zeros
        for f in range(_F):
            wsum = wsum + wbuf[pl.ds(f * _RC + h * _LANES, _LANES)]

        def row_body(i, ovec, h=h):
            acc0 = zeros
            acc1 = zeros
            q0 = zeros
            q1 = zeros
            for f in range(_F):
                row = f * _RC + h * _LANES + i
                v0 = gbuf[row, pl.ds(0, _LANES)]
                v1 = gbuf[row, pl.ds(_LANES, _LANES)]
                acc0 = acc0 + v0
                q0 = q0 + v0 * v0
                acc1 = acc1 + v1
                q1 = q1 + v1 * v1
            d = (acc0 * acc0 - q0) + (acc1 * acc1 - q1)
            t = jnp.sum(d) * jnp.float32(0.5)
            return jnp.where(lanes == i, t, ovec)

        ovec = lax.fori_loop(0, _LANES, row_body, zeros)
        outv[pl.ds(c * _RC + h * _LANES, _LANES)] = ovec + wsum


def _fm_body(xt_hbm, V_hbm, w_hbm, out_hbm,
             idx_v, g0, g1, wb0, wb1, outv, sem0, sem1):
    info = plsc.get_sparse_core_info()
    wid = lax.axis_index("s") * info.num_cores + lax.axis_index("c")
    pltpu.sync_copy(xt_hbm.at[:, pl.ds(wid * _RW, _RW)], idx_v)
    _issue(0, idx_v, V_hbm, w_hbm, g0, wb0, sem0)

    def body(s, carry):
        c0 = 2 * s
        c1 = 2 * s + 1
        _issue(c1, idx_v, V_hbm, w_hbm, g1, wb1, sem1)
        _drain(c0, idx_v, V_hbm, w_hbm, g0, wb0, sem0)
        _compute(c0, g0, wb0, outv)

        @pl.when(s < _NCH // 2 - 1)
        def _():
            _issue(c0 + 2, idx_v, V_hbm, w_hbm, g0, wb0, sem0)

        _drain(c1, idx_v, V_hbm, w_hbm, g1, wb1, sem1)
        _compute(c1, g1, wb1, outv)
        return carry

    lax.fori_loop(0, _NCH // 2, body, 0)
    pltpu.sync_copy(outv, out_hbm.at[pl.ds(wid * _RW, _RW)])


@jax.jit
def _fm(xt, V, w):
    mesh = plsc.VectorSubcoreMesh(core_axis_name="c", subcore_axis_name="s")
    run = functools.partial(
        pl.kernel,
        out_type=jax.ShapeDtypeStruct((_B,), jnp.float32),
        mesh=mesh,
        compiler_params=pltpu.CompilerParams(
            needs_layout_passes=False, use_tc_tiling_on_sc=False),
        scratch_types=[
            pltpu.VMEM((_F, _RW), jnp.int32),
            pltpu.VMEM((_F * _RC, _K), jnp.float32),
            pltpu.VMEM((_F * _RC, _K), jnp.float32),
            pltpu.VMEM((_F * _RC,), jnp.float32),
            pltpu.VMEM((_F * _RC,), jnp.float32),
            pltpu.VMEM((_RW,), jnp.float32),
            pltpu.SemaphoreType.DMA,
            pltpu.SemaphoreType.DMA,
        ],
    )(_fm_body)
    return run(xt, V, w)


def kernel(x, V, w, w0):
    return _fm(x.T, V, w) + w0
